# baseline (speedup 1.0000x reference)
import jax
import jax.numpy as jnp
from jax import lax
from jax.experimental import pallas as pl
from jax.experimental.pallas import tpu as pltpu
from jax.experimental.pallas import tpu_sc as plsc

N_TET_V = 100000
NV = 50000
NF = 100000

NS = 16
L = 16

TC_ = 6256
NT_PAD = NS * TC_

VC = 3200
NV_PAD = NS * VC
VB = VC // 128

FC = 3328
NF_PAD = 32 * FC
FB = FC // 128


def _body(tx_hbm, ty_hbm, tz_hbm, vid_hbm, f0_hbm, f1_hbm, f2_hbm, zer_hbm,
          px_o, py_o, pz_o, a0x_o, a0y_o, a0z_o, a1x_o, a1y_o, a1z_o,
          tb_v, vid_v, px_v, py_v, pz_v,
          f0_v, f1_v, f2_v, bA0, bA1, bA2, bB0, bB1, bB2,
          v0x, v0y, v0z, v1x, v1y, v1z, v2x, v2y, v2z,
          cx_v, cy_v, cz_v, ax_v, ay_v, az_v,
          stx, sty, stz, spx, spy, spz, sax, say, saz,
          sem, semA, semB):
    c = lax.axis_index("c")
    s = lax.axis_index("s")

    scope0 = jax.named_scope("ph0_stage")
    scope0.__enter__()
    for sh in (sax, say, saz):
        pltpu.sync_copy(zer_hbm, sh.at[pl.ds(s * VC, VC)])

    slt = pl.ds(s * TC_, TC_)
    for th, st in ((tx_hbm, stx), (ty_hbm, sty), (tz_hbm, stz)):
        pltpu.sync_copy(th.at[slt], tb_v)
        pltpu.sync_copy(tb_v, st.at[slt])

    pltpu.sync_copy(vid_hbm.at[pl.ds(s * VC, VC)], vid_v)

    wid = c * NS + s
    for fh, fv in ((f0_hbm, f0_v), (f1_hbm, f1_v), (f2_hbm, f2_v)):
        pltpu.sync_copy(fh.at[pl.ds(wid * FC, FC)], fv)

    plsc.subcore_barrier()
    scope0.__exit__(None, None, None)

    scope1 = jax.named_scope("ph1_vpos")
    scope1.__enter__()
    for st, dst in ((stx, px_v), (sty, py_v), (stz, pz_v)):
        for j in range(VB):
            sl = pl.ds(j * 128, 128)
            pltpu.async_copy(st.at[vid_v.at[sl]], dst.at[sl], sem)
    for st, dst in ((stx, px_v), (sty, py_v), (stz, pz_v)):
        for j in range(VB):
            sl = pl.ds(j * 128, 128)
            pltpu.make_async_copy(st.at[vid_v.at[sl]], dst.at[sl], sem).wait()

    for sh, src in ((spx, px_v), (spy, py_v), (spz, pz_v)):
        pltpu.sync_copy(src, sh.at[pl.ds(s * VC, VC)])

    @pl.when(c == 0)
    def _():
        for dst, src in ((px_o, px_v), (py_o, py_v), (pz_o, pz_v)):
            pltpu.sync_copy(src, dst.at[pl.ds(s * VC, VC)])

    plsc.subcore_barrier()
    scope1.__exit__(None, None, None)

    scope2 = jax.named_scope("ph2_faces")
    scope2.__enter__()
    trips = ((f0_v, (v0x, v0y, v0z)),
             (f1_v, (v1x, v1y, v1z)),
             (f2_v, (v2x, v2y, v2z)))

    def fire_gathers(j, carry):
        sl = pl.ds(j * 128, 128)
        for fv, dsts in trips:
            for sh, dst in zip((spx, spy, spz), dsts):
                pltpu.async_copy(sh.at[fv.at[sl]], dst.at[sl], sem)
        return carry

    def drain_gathers(j, carry):
        sl = pl.ds(j * 128, 128)
        for fv, dsts in trips:
            for sh, dst in zip((spx, spy, spz), dsts):
                pltpu.make_async_copy(sh.at[fv.at[sl]], dst.at[sl], sem).wait()
        return carry

    lax.fori_loop(0, FB, fire_gathers, 0)
    lax.fori_loop(0, FB, drain_gathers, 0)

    def drain_adds(bsem):
        for _ in range(9):
            pltpu.make_async_copy(cx_v.at[pl.ds(0, 128)], sax.at[bA0], bsem).wait()

    def one_batch(b, bufs, bsem):
        b0_v, b1_v, b2_v = bufs
        for fv, bv in ((f0_v, b0_v), (f1_v, b1_v), (f2_v, b2_v)):
            for q in range(128 // L):
                bv[pl.ds(q * L, L)] = fv[pl.ds(b * 128 + q * L, L)]
        for g in range(128 // L):
            so = pl.ds(b * 128 + g * L, L)
            ax, ay, az = v0x[so], v0y[so], v0z[so]
            e1x, e1y, e1z = v1x[so] - ax, v1y[so] - ay, v1z[so] - az
            e2x, e2y, e2z = v2x[so] - ax, v2y[so] - ay, v2z[so] - az
            cx_v[so] = e1y * e2z - e1z * e2y
            cy_v[so] = e1z * e2x - e1x * e2z
            cz_v[so] = e1x * e2y - e1y * e2x
        slb = pl.ds(b * 128, 128)
        for bv in (b0_v, b1_v, b2_v):
            for sh, src in ((sax, cx_v), (say, cy_v), (saz, cz_v)):
                pltpu.async_copy(src.at[slb], sh.at[bv], bsem, add=True)

    def face_pair(t, carry):
        @pl.when(t > 0)
        def _():
            drain_adds(semA)
        one_batch(2 * t, (bA0, bA1, bA2), semA)

        @pl.when(t > 0)
        def _():
            drain_adds(semB)
        one_batch(2 * t + 1, (bB0, bB1, bB2), semB)
        return carry

    lax.fori_loop(0, FB // 2, face_pair, 0)
    drain_adds(semA)
    drain_adds(semB)
    plsc.subcore_barrier()
    scope2.__exit__(None, None, None)

    scope3 = jax.named_scope("ph3_out")
    scope3.__enter__()
    sl3 = pl.ds(s * VC, VC)

    @pl.when(c == 0)
    def _():
        for sh, buf, dst in ((sax, ax_v, a0x_o), (say, ay_v, a0y_o), (saz, az_v, a0z_o)):
            pltpu.sync_copy(sh.at[sl3], buf)
            pltpu.sync_copy(buf, dst.at[sl3])

    @pl.when(c == 1)
    def _():
        for sh, buf, dst in ((sax, ax_v, a1x_o), (say, ay_v, a1y_o), (saz, az_v, a1z_o)):
            pltpu.sync_copy(sh.at[sl3], buf)
            pltpu.sync_copy(buf, dst.at[sl3])

    scope3.__exit__(None, None, None)


def _norm_tc(a0x, a0y, a0z, a1x, a1y, a1z, ox, oy, oz):
    x = a0x[...] + a1x[...]
    y = a0y[...] + a1y[...]
    z = a0z[...] + a1z[...]
    d = x * x + y * y + z * z
    ok = d > 1e-20
    n = jnp.maximum(jnp.sqrt(d), 1e-12)
    ox[...] = jnp.where(ok, x / n, 0.0)
    oy[...] = jnp.where(ok, y / n, 0.0)
    oz[...] = jnp.where(ok, z / n, 1.0)


@jax.jit
def kernel(tet_v, surface_vid, surface_f):
    tpad = jnp.zeros((NT_PAD - N_TET_V,), jnp.float32)
    tx = jnp.concatenate([tet_v[:, 0], tpad])
    ty = jnp.concatenate([tet_v[:, 1], tpad])
    tz = jnp.concatenate([tet_v[:, 2], tpad])

    vid = surface_vid.astype(jnp.int32)
    vid = jnp.concatenate([vid, jnp.zeros((NV_PAD - NV,), jnp.int32)])

    f32i = surface_f.astype(jnp.int32)
    pad = jnp.full((NF_PAD - NF,), NV, jnp.int32)
    f0 = jnp.concatenate([f32i[:, 0], pad])
    f1 = jnp.concatenate([f32i[:, 1], pad])
    f2 = jnp.concatenate([f32i[:, 2], pad])

    zer = jnp.zeros((VC,), jnp.float32)

    plane = jax.ShapeDtypeStruct((NV_PAD,), jnp.float32)
    vmemf = lambda n: pltpu.VMEM((n,), jnp.float32)
    vmemi = lambda n: pltpu.VMEM((n,), jnp.int32)
    shmf = lambda n: pltpu.VMEM_SHARED((n,), jnp.float32)
    run = pl.kernel(
        _body,
        out_type=(plane,) * 9,
        mesh=plsc.VectorSubcoreMesh(core_axis_name="c", subcore_axis_name="s"),
        scratch_types=[
            vmemf(TC_), vmemi(VC),
            vmemf(VC), vmemf(VC), vmemf(VC),
            vmemi(FC), vmemi(FC), vmemi(FC),
            vmemi(128), vmemi(128), vmemi(128),
            vmemi(128), vmemi(128), vmemi(128),
            vmemf(FC), vmemf(FC), vmemf(FC),
            vmemf(FC), vmemf(FC), vmemf(FC),
            vmemf(FC), vmemf(FC), vmemf(FC),
            vmemf(FC), vmemf(FC), vmemf(FC),
            vmemf(VC), vmemf(VC), vmemf(VC),
            shmf(NT_PAD), shmf(NT_PAD), shmf(NT_PAD),
            shmf(NV_PAD), shmf(NV_PAD), shmf(NV_PAD),
            shmf(NV_PAD), shmf(NV_PAD), shmf(NV_PAD),
            pltpu.SemaphoreType.DMA,
            pltpu.SemaphoreType.DMA,
            pltpu.SemaphoreType.DMA,
        ],
    )
    px, py, pz, a0x, a0y, a0z, a1x, a1y, a1z = run(
        tx, ty, tz, vid, f0, f1, f2, zer)

    blk = jax.ShapeDtypeStruct((NV_PAD // 128, 128), jnp.float32)
    nx, ny, nz = pl.pallas_call(
        _norm_tc,
        out_shape=(blk,) * 3,
    )(*(a.reshape(NV_PAD // 128, 128) for a in (a0x, a0y, a0z, a1x, a1y, a1z)))

    v_pos = jnp.stack([px[:NV], py[:NV], pz[:NV]], axis=1)
    v_nrm = jnp.stack([nx.reshape(-1)[:NV],
                       ny.reshape(-1)[:NV],
                       nz.reshape(-1)[:NV]], axis=1)
    return v_pos, v_nrm


# speedup vs baseline: 1.1463x; 1.1463x over previous
import jax
import jax.numpy as jnp
from jax import lax
from jax.experimental import pallas as pl
from jax.experimental.pallas import tpu as pltpu
from jax.experimental.pallas import tpu_sc as plsc

N_TET_V = 100000
NV = 50000
NF = 100000

NS = 16
L = 16

TC_ = 6256
NT_PAD = NS * TC_

VC = 3200
NV_PAD = NS * VC
VB = VC // 128

FC = 3328
NF_PAD = 32 * FC
FB = FC // 128


def _body(tx_hbm, ty_hbm, tz_hbm, vid_hbm, f0_hbm, f1_hbm, f2_hbm, zer_hbm,
          px_o, py_o, pz_o, a0x_o, a0y_o, a0z_o, a1x_o, a1y_o, a1z_o,
          tb_v, vid_v, px_v, py_v, pz_v,
          f0_v, f1_v, f2_v, bA0, bA1, bA2, bB0, bB1, bB2,
          gA0x, gA0y, gA0z, gA1x, gA1y, gA1z, gA2x, gA2y, gA2z,
          gB0x, gB0y, gB0z, gB1x, gB1y, gB1z, gB2x, gB2y, gB2z,
          cx_v, cy_v, cz_v, ax_v, ay_v, az_v,
          stx, sty, stz, spx, spy, spz, sax, say, saz,
          sem, semA, semB, semGA, semGB):
    c = lax.axis_index("c")
    s = lax.axis_index("s")

    scope0 = jax.named_scope("ph0_stage")
    scope0.__enter__()
    for sh in (sax, say, saz):
        pltpu.sync_copy(zer_hbm, sh.at[pl.ds(s * VC, VC)])

    slt = pl.ds(s * TC_, TC_)
    for th, st in ((tx_hbm, stx), (ty_hbm, sty), (tz_hbm, stz)):
        pltpu.sync_copy(th.at[slt], tb_v)
        pltpu.sync_copy(tb_v, st.at[slt])

    pltpu.sync_copy(vid_hbm.at[pl.ds(s * VC, VC)], vid_v)

    wid = c * NS + s
    for fh, fv in ((f0_hbm, f0_v), (f1_hbm, f1_v), (f2_hbm, f2_v)):
        pltpu.sync_copy(fh.at[pl.ds(wid * FC, FC)], fv)

    plsc.subcore_barrier()
    scope0.__exit__(None, None, None)

    scope1 = jax.named_scope("ph1_vpos")
    scope1.__enter__()
    for st, dst in ((stx, px_v), (sty, py_v), (stz, pz_v)):
        for j in range(VB):
            sl = pl.ds(j * 128, 128)
            pltpu.async_copy(st.at[vid_v.at[sl]], dst.at[sl], sem)
    for st, dst in ((stx, px_v), (sty, py_v), (stz, pz_v)):
        for j in range(VB):
            sl = pl.ds(j * 128, 128)
            pltpu.make_async_copy(st.at[vid_v.at[sl]], dst.at[sl], sem).wait()

    for sh, src in ((spx, px_v), (spy, py_v), (spz, pz_v)):
        pltpu.sync_copy(src, sh.at[pl.ds(s * VC, VC)])

    @pl.when(c == 0)
    def _():
        for dst, src in ((px_o, px_v), (py_o, py_v), (pz_o, pz_v)):
            pltpu.sync_copy(src, dst.at[pl.ds(s * VC, VC)])

    plsc.subcore_barrier()
    scope1.__exit__(None, None, None)

    scope2 = jax.named_scope("ph2_faces")
    scope2.__enter__()
    VA = ((gA0x, gA0y, gA0z), (gA1x, gA1y, gA1z), (gA2x, gA2y, gA2z))
    VB_ = ((gB0x, gB0y, gB0z), (gB1x, gB1y, gB1z), (gB2x, gB2y, gB2z))
    FPL = (f0_v, f1_v, f2_v)
    SPL = (spx, spy, spz)

    def fire_g(b, vset, gsem):
        sl = pl.ds(b * 128, 128)
        for fv, dsts in zip(FPL, vset):
            for sh, dst in zip(SPL, dsts):
                pltpu.async_copy(sh.at[fv.at[sl]], dst, gsem)

    def drain_g(b, vset, gsem):
        sl = pl.ds(b * 128, 128)
        for fv, dsts in zip(FPL, vset):
            for sh, dst in zip(SPL, dsts):
                pltpu.make_async_copy(sh.at[fv.at[sl]], dst, gsem).wait()

    def drain_adds(bsem):
        for _ in range(9):
            pltpu.make_async_copy(cx_v.at[pl.ds(0, 128)], sax.at[bA0], bsem).wait()

    def one_batch(b, vset, bufs, bsem):
        b0_v, b1_v, b2_v = bufs
        for fv, bv in ((f0_v, b0_v), (f1_v, b1_v), (f2_v, b2_v)):
            for q in range(128 // L):
                bv[pl.ds(q * L, L)] = fv[pl.ds(b * 128 + q * L, L)]
        (v0x, v0y, v0z), (v1x, v1y, v1z), (v2x, v2y, v2z) = vset
        for g in range(128 // L):
            sl = pl.ds(g * L, L)
            so = pl.ds(b * 128 + g * L, L)
            ax, ay, az = v0x[sl], v0y[sl], v0z[sl]
            e1x, e1y, e1z = v1x[sl] - ax, v1y[sl] - ay, v1z[sl] - az
            e2x, e2y, e2z = v2x[sl] - ax, v2y[sl] - ay, v2z[sl] - az
            cx_v[so] = e1y * e2z - e1z * e2y
            cy_v[so] = e1z * e2x - e1x * e2z
            cz_v[so] = e1x * e2y - e1y * e2x
        slb = pl.ds(b * 128, 128)
        for bv in (b0_v, b1_v, b2_v):
            for sh, src in ((sax, cx_v), (say, cy_v), (saz, cz_v)):
                pltpu.async_copy(src.at[slb], sh.at[bv], bsem, add=True)

    T = FB // 2
    fire_g(0, VA, semGA)
    fire_g(1, VB_, semGB)

    def face_pair(t, carry):
        drain_g(2 * t, VA, semGA)

        @pl.when(t + 1 < T)
        def _():
            fire_g(2 * t + 2, VA, semGA)

        @pl.when(t > 0)
        def _():
            drain_adds(semA)
        one_batch(2 * t, VA, (bA0, bA1, bA2), semA)

        drain_g(2 * t + 1, VB_, semGB)

        @pl.when(t + 1 < T)
        def _():
            fire_g(2 * t + 3, VB_, semGB)

        @pl.when(t > 0)
        def _():
            drain_adds(semB)
        one_batch(2 * t + 1, VB_, (bB0, bB1, bB2), semB)
        return carry

    lax.fori_loop(0, T, face_pair, 0)
    drain_adds(semA)
    drain_adds(semB)
    plsc.subcore_barrier()
    scope2.__exit__(None, None, None)

    scope3 = jax.named_scope("ph3_out")
    scope3.__enter__()
    sl3 = pl.ds(s * VC, VC)

    @pl.when(c == 0)
    def _():
        for sh, buf, dst in ((sax, ax_v, a0x_o), (say, ay_v, a0y_o), (saz, az_v, a0z_o)):
            pltpu.sync_copy(sh.at[sl3], buf)
            pltpu.sync_copy(buf, dst.at[sl3])

    @pl.when(c == 1)
    def _():
        for sh, buf, dst in ((sax, ax_v, a1x_o), (say, ay_v, a1y_o), (saz, az_v, a1z_o)):
            pltpu.sync_copy(sh.at[sl3], buf)
            pltpu.sync_copy(buf, dst.at[sl3])

    scope3.__exit__(None, None, None)


def _norm_tc(a0x, a0y, a0z, a1x, a1y, a1z, ox, oy, oz):
    x = a0x[...] + a1x[...]
    y = a0y[...] + a1y[...]
    z = a0z[...] + a1z[...]
    d = x * x + y * y + z * z
    ok = d > 1e-20
    n = jnp.maximum(jnp.sqrt(d), 1e-12)
    ox[...] = jnp.where(ok, x / n, 0.0)
    oy[...] = jnp.where(ok, y / n, 0.0)
    oz[...] = jnp.where(ok, z / n, 1.0)


@jax.jit
def kernel(tet_v, surface_vid, surface_f):
    tpad = jnp.zeros((NT_PAD - N_TET_V,), jnp.float32)
    tx = jnp.concatenate([tet_v[:, 0], tpad])
    ty = jnp.concatenate([tet_v[:, 1], tpad])
    tz = jnp.concatenate([tet_v[:, 2], tpad])

    vid = surface_vid.astype(jnp.int32)
    vid = jnp.concatenate([vid, jnp.zeros((NV_PAD - NV,), jnp.int32)])

    f32i = surface_f.astype(jnp.int32)
    pad = jnp.full((NF_PAD - NF,), NV, jnp.int32)
    f0 = jnp.concatenate([f32i[:, 0], pad])
    f1 = jnp.concatenate([f32i[:, 1], pad])
    f2 = jnp.concatenate([f32i[:, 2], pad])

    zer = jnp.zeros((VC,), jnp.float32)

    plane = jax.ShapeDtypeStruct((NV_PAD,), jnp.float32)
    vmemf = lambda n: pltpu.VMEM((n,), jnp.float32)
    vmemi = lambda n: pltpu.VMEM((n,), jnp.int32)
    shmf = lambda n: pltpu.VMEM_SHARED((n,), jnp.float32)
    run = pl.kernel(
        _body,
        out_type=(plane,) * 9,
        mesh=plsc.VectorSubcoreMesh(core_axis_name="c", subcore_axis_name="s"),
        scratch_types=[
            vmemf(TC_), vmemi(VC),
            vmemf(VC), vmemf(VC), vmemf(VC),
            vmemi(FC), vmemi(FC), vmemi(FC),
            vmemi(128), vmemi(128), vmemi(128),
            vmemi(128), vmemi(128), vmemi(128),
            vmemf(128), vmemf(128), vmemf(128),
            vmemf(128), vmemf(128), vmemf(128),
            vmemf(128), vmemf(128), vmemf(128),
            vmemf(128), vmemf(128), vmemf(128),
            vmemf(128), vmemf(128), vmemf(128),
            vmemf(128), vmemf(128), vmemf(128),
            vmemf(FC), vmemf(FC), vmemf(FC),
            vmemf(VC), vmemf(VC), vmemf(VC),
            shmf(NT_PAD), shmf(NT_PAD), shmf(NT_PAD),
            shmf(NV_PAD), shmf(NV_PAD), shmf(NV_PAD),
            shmf(NV_PAD), shmf(NV_PAD), shmf(NV_PAD),
            pltpu.SemaphoreType.DMA,
            pltpu.SemaphoreType.DMA,
            pltpu.SemaphoreType.DMA,
            pltpu.SemaphoreType.DMA,
            pltpu.SemaphoreType.DMA,
        ],
    )
    px, py, pz, a0x, a0y, a0z, a1x, a1y, a1z = run(
        tx, ty, tz, vid, f0, f1, f2, zer)

    blk = jax.ShapeDtypeStruct((NV_PAD // 128, 128), jnp.float32)
    nx, ny, nz = pl.pallas_call(
        _norm_tc,
        out_shape=(blk,) * 3,
    )(*(a.reshape(NV_PAD // 128, 128) for a in (a0x, a0y, a0z, a1x, a1y, a1z)))

    v_pos = jnp.stack([px[:NV], py[:NV], pz[:NV]], axis=1)
    v_nrm = jnp.stack([nx.reshape(-1)[:NV],
                       ny.reshape(-1)[:NV],
                       nz.reshape(-1)[:NV]], axis=1)
    return v_pos, v_nrm


# speedup vs baseline: 1.1819x; 1.0310x over previous
import jax
import jax.numpy as jnp
from jax import lax
from jax.experimental import pallas as pl
from jax.experimental.pallas import tpu as pltpu
from jax.experimental.pallas import tpu_sc as plsc

N_TET_V = 100000
NV = 50000
NF = 100000

NS = 16
L = 16

TC_ = 6256
NT_PAD = NS * TC_

VC = 3200
NV_PAD = NS * VC
VB = VC // 128

FC = 3328
NF_PAD = 32 * FC
FB = FC // 128


def _body(tx_hbm, ty_hbm, tz_hbm, vid_hbm, f0_hbm, f1_hbm, f2_hbm, zer_hbm,
          px_o, py_o, pz_o, a0x_o, a0y_o, a0z_o, a1x_o, a1y_o, a1z_o,
          tb_v, vid_v, px_v, py_v, pz_v,
          f0_v, f1_v, f2_v, bA0, bA1, bA2, bB0, bB1, bB2,
          gA0x, gA0y, gA0z, gA1x, gA1y, gA1z, gA2x, gA2y, gA2z,
          gB0x, gB0y, gB0z, gB1x, gB1y, gB1z, gB2x, gB2y, gB2z,
          cx_v, cy_v, cz_v, ax_v, ay_v, az_v,
          stx, sty, stz, spx, spy, spz, sax, say, saz,
          sem, semA, semB, semGA, semGB):
    c = lax.axis_index("c")
    s = lax.axis_index("s")

    scope0 = jax.named_scope("ph0_stage")
    scope0.__enter__()
    for sh in (sax, say, saz):
        pltpu.sync_copy(zer_hbm, sh.at[pl.ds(s * VC, VC)])

    slt = pl.ds(s * TC_, TC_)
    for th, st in ((tx_hbm, stx), (ty_hbm, sty), (tz_hbm, stz)):
        pltpu.sync_copy(th.at[slt], tb_v)
        pltpu.sync_copy(tb_v, st.at[slt])

    pltpu.sync_copy(vid_hbm.at[pl.ds(s * VC, VC)], vid_v)

    wid = c * NS + s
    for fh, fv in ((f0_hbm, f0_v), (f1_hbm, f1_v), (f2_hbm, f2_v)):
        pltpu.sync_copy(fh.at[pl.ds(wid * FC, FC)], fv)

    plsc.subcore_barrier()
    scope0.__exit__(None, None, None)

    scope1 = jax.named_scope("ph1_vpos")
    scope1.__enter__()
    for st, dst in ((stx, px_v), (sty, py_v), (stz, pz_v)):
        for j in range(VB):
            sl = pl.ds(j * 128, 128)
            pltpu.async_copy(st.at[vid_v.at[sl]], dst.at[sl], sem)
    for st, dst in ((stx, px_v), (sty, py_v), (stz, pz_v)):
        for j in range(VB):
            sl = pl.ds(j * 128, 128)
            pltpu.make_async_copy(st.at[vid_v.at[sl]], dst.at[sl], sem).wait()

    for sh, src in ((spx, px_v), (spy, py_v), (spz, pz_v)):
        pltpu.sync_copy(src, sh.at[pl.ds(s * VC, VC)])

    @pl.when(c == 0)
    def _():
        for dst, src in ((px_o, px_v), (py_o, py_v), (pz_o, pz_v)):
            pltpu.sync_copy(src, dst.at[pl.ds(s * VC, VC)])

    plsc.subcore_barrier()
    scope1.__exit__(None, None, None)

    scope2 = jax.named_scope("ph2_faces")
    scope2.__enter__()
    VA = ((gA0x, gA0y, gA0z), (gA1x, gA1y, gA1z), (gA2x, gA2y, gA2z))
    VB_ = ((gB0x, gB0y, gB0z), (gB1x, gB1y, gB1z), (gB2x, gB2y, gB2z))
    FPL = (f0_v, f1_v, f2_v)
    SPL = (spx, spy, spz)

    def fire_g(b, vset, gsem):
        sl = pl.ds(b * 128, 128)
        for fv, dsts in zip(FPL, vset):
            for sh, dst in zip(SPL, dsts):
                pltpu.async_copy(sh.at[fv.at[sl]], dst, gsem)

    def drain_g(b, vset, gsem):
        sl = pl.ds(b * 128, 128)
        for fv, dsts in zip(FPL, vset):
            for sh, dst in zip(SPL, dsts):
                pltpu.make_async_copy(sh.at[fv.at[sl]], dst, gsem).wait()

    def drain_adds(bsem):
        for _ in range(9):
            pltpu.make_async_copy(cx_v.at[pl.ds(0, 128)], sax.at[bA0], bsem).wait()

    def one_batch(b, vset, bufs, bsem):
        b0_v, b1_v, b2_v = bufs
        for fv, bv in ((f0_v, b0_v), (f1_v, b1_v), (f2_v, b2_v)):
            for q in range(128 // L):
                bv[pl.ds(q * L, L)] = fv[pl.ds(b * 128 + q * L, L)]
        (v0x, v0y, v0z), (v1x, v1y, v1z), (v2x, v2y, v2z) = vset
        for g in range(128 // L):
            sl = pl.ds(g * L, L)
            so = pl.ds(b * 128 + g * L, L)
            ax, ay, az = v0x[sl], v0y[sl], v0z[sl]
            e1x, e1y, e1z = v1x[sl] - ax, v1y[sl] - ay, v1z[sl] - az
            e2x, e2y, e2z = v2x[sl] - ax, v2y[sl] - ay, v2z[sl] - az
            cx_v[so] = e1y * e2z - e1z * e2y
            cy_v[so] = e1z * e2x - e1x * e2z
            cz_v[so] = e1x * e2y - e1y * e2x
        slb = pl.ds(b * 128, 128)
        for bv in (b0_v, b1_v, b2_v):
            for sh, src in ((sax, cx_v), (say, cy_v), (saz, cz_v)):
                pltpu.async_copy(src.at[slb], sh.at[bv], bsem, add=True)

    T = FB // 2
    fire_g(0, VA, semGA)
    fire_g(1, VB_, semGB)

    def face_pair(t, carry):
        drain_g(2 * t, VA, semGA)

        @pl.when(t > 0)
        def _():
            drain_adds(semA)
        one_batch(2 * t, VA, (bA0, bA1, bA2), semA)

        @pl.when(t + 1 < T)
        def _():
            fire_g(2 * t + 2, VA, semGA)

        drain_g(2 * t + 1, VB_, semGB)

        @pl.when(t > 0)
        def _():
            drain_adds(semB)
        one_batch(2 * t + 1, VB_, (bB0, bB1, bB2), semB)

        @pl.when(t + 1 < T)
        def _():
            fire_g(2 * t + 3, VB_, semGB)
        return carry

    lax.fori_loop(0, T, face_pair, 0)
    drain_adds(semA)
    drain_adds(semB)
    plsc.subcore_barrier()
    scope2.__exit__(None, None, None)

    scope3 = jax.named_scope("ph3_out")
    scope3.__enter__()
    sl3 = pl.ds(s * VC, VC)

    @pl.when(c == 0)
    def _():
        for sh, buf, dst in ((sax, ax_v, a0x_o), (say, ay_v, a0y_o), (saz, az_v, a0z_o)):
            pltpu.sync_copy(sh.at[sl3], buf)
            pltpu.sync_copy(buf, dst.at[sl3])

    @pl.when(c == 1)
    def _():
        for sh, buf, dst in ((sax, ax_v, a1x_o), (say, ay_v, a1y_o), (saz, az_v, a1z_o)):
            pltpu.sync_copy(sh.at[sl3], buf)
            pltpu.sync_copy(buf, dst.at[sl3])

    scope3.__exit__(None, None, None)


def _norm_tc(a0x, a0y, a0z, a1x, a1y, a1z, ox, oy, oz):
    x = a0x[...] + a1x[...]
    y = a0y[...] + a1y[...]
    z = a0z[...] + a1z[...]
    d = x * x + y * y + z * z
    ok = d > 1e-20
    n = jnp.maximum(jnp.sqrt(d), 1e-12)
    ox[...] = jnp.where(ok, x / n, 0.0)
    oy[...] = jnp.where(ok, y / n, 0.0)
    oz[...] = jnp.where(ok, z / n, 1.0)


@jax.jit
def kernel(tet_v, surface_vid, surface_f):
    tpad = jnp.zeros((NT_PAD - N_TET_V,), jnp.float32)
    tx = jnp.concatenate([tet_v[:, 0], tpad])
    ty = jnp.concatenate([tet_v[:, 1], tpad])
    tz = jnp.concatenate([tet_v[:, 2], tpad])

    vid = surface_vid.astype(jnp.int32)
    vid = jnp.concatenate([vid, jnp.zeros((NV_PAD - NV,), jnp.int32)])

    f32i = surface_f.astype(jnp.int32)
    pad = jnp.full((NF_PAD - NF,), NV, jnp.int32)
    f0 = jnp.concatenate([f32i[:, 0], pad])
    f1 = jnp.concatenate([f32i[:, 1], pad])
    f2 = jnp.concatenate([f32i[:, 2], pad])

    zer = jnp.zeros((VC,), jnp.float32)

    plane = jax.ShapeDtypeStruct((NV_PAD,), jnp.float32)
    vmemf = lambda n: pltpu.VMEM((n,), jnp.float32)
    vmemi = lambda n: pltpu.VMEM((n,), jnp.int32)
    shmf = lambda n: pltpu.VMEM_SHARED((n,), jnp.float32)
    run = pl.kernel(
        _body,
        out_type=(plane,) * 9,
        mesh=plsc.VectorSubcoreMesh(core_axis_name="c", subcore_axis_name="s"),
        scratch_types=[
            vmemf(TC_), vmemi(VC),
            vmemf(VC), vmemf(VC), vmemf(VC),
            vmemi(FC), vmemi(FC), vmemi(FC),
            vmemi(128), vmemi(128), vmemi(128),
            vmemi(128), vmemi(128), vmemi(128),
            vmemf(128), vmemf(128), vmemf(128),
            vmemf(128), vmemf(128), vmemf(128),
            vmemf(128), vmemf(128), vmemf(128),
            vmemf(128), vmemf(128), vmemf(128),
            vmemf(128), vmemf(128), vmemf(128),
            vmemf(128), vmemf(128), vmemf(128),
            vmemf(FC), vmemf(FC), vmemf(FC),
            vmemf(VC), vmemf(VC), vmemf(VC),
            shmf(NT_PAD), shmf(NT_PAD), shmf(NT_PAD),
            shmf(NV_PAD), shmf(NV_PAD), shmf(NV_PAD),
            shmf(NV_PAD), shmf(NV_PAD), shmf(NV_PAD),
            pltpu.SemaphoreType.DMA,
            pltpu.SemaphoreType.DMA,
            pltpu.SemaphoreType.DMA,
            pltpu.SemaphoreType.DMA,
            pltpu.SemaphoreType.DMA,
        ],
    )
    px, py, pz, a0x, a0y, a0z, a1x, a1y, a1z = run(
        tx, ty, tz, vid, f0, f1, f2, zer)

    blk = jax.ShapeDtypeStruct((NV_PAD // 128, 128), jnp.float32)
    nx, ny, nz = pl.pallas_call(
        _norm_tc,
        out_shape=(blk,) * 3,
    )(*(a.reshape(NV_PAD // 128, 128) for a in (a0x, a0y, a0z, a1x, a1y, a1z)))

    v_pos = jnp.stack([px[:NV], py[:NV], pz[:NV]], axis=1)
    v_nrm = jnp.stack([nx.reshape(-1)[:NV],
                       ny.reshape(-1)[:NV],
                       nz.reshape(-1)[:NV]], axis=1)
    return v_pos, v_nrm


# speedup vs baseline: 1.5331x; 1.2972x over previous
import jax
import jax.numpy as jnp
from jax import lax
from jax.experimental import pallas as pl
from jax.experimental.pallas import tpu as pltpu
from jax.experimental.pallas import tpu_sc as plsc

N_TET_V = 100000
NV = 50000
NF = 100000

NS = 16
L = 16

TC_ = 6256
NT_PAD = NS * TC_

VC = 3200
NV_PAD = NS * VC
VB = VC // 128

FC = 3200
NF_PAD = 32 * FC
FB = FC // 128


def _body(tx_hbm, ty_hbm, tz_hbm, vid_hbm, f0_hbm, f1_hbm, f2_hbm, zer_hbm,
          px_o, py_o, pz_o, a0x_o, a0y_o, a0z_o, a1x_o, a1y_o, a1z_o,
          tb_v, vid_v, px_v, py_v, pz_v,
          f0_v, f1_v, f2_v, bA0, bA1, bA2, bB0, bB1, bB2,
          v0x, v0y, v0z, v1x, v1y, v1z, v2x, v2y, v2z,
          cx_v, cy_v, cz_v, ax_v, ay_v, az_v,
          stx, sty, stz, spx, spy, spz, sax, say, saz,
          sem, semA, semB):
    c = lax.axis_index("c")
    s = lax.axis_index("s")

    scope0 = jax.named_scope("ph0_stage")
    scope0.__enter__()
    for sh in (sax, say, saz):
        pltpu.sync_copy(zer_hbm, sh.at[pl.ds(s * VC, VC)])

    slt = pl.ds(s * TC_, TC_)
    for th, st in ((tx_hbm, stx), (ty_hbm, sty), (tz_hbm, stz)):
        pltpu.sync_copy(th.at[slt], tb_v)
        pltpu.sync_copy(tb_v, st.at[slt])

    pltpu.sync_copy(vid_hbm.at[pl.ds(s * VC, VC)], vid_v)

    wid = c * NS + s
    for fh, fv in ((f0_hbm, f0_v), (f1_hbm, f1_v), (f2_hbm, f2_v)):
        pltpu.sync_copy(fh.at[pl.ds(wid * FC, FC)], fv)

    plsc.subcore_barrier()
    scope0.__exit__(None, None, None)

    scope1 = jax.named_scope("ph1_vpos")
    scope1.__enter__()
    for st, dst in ((stx, px_v), (sty, py_v), (stz, pz_v)):
        for j in range(VB):
            sl = pl.ds(j * 128, 128)
            pltpu.async_copy(st.at[vid_v.at[sl]], dst.at[sl], sem)
    for st, dst in ((stx, px_v), (sty, py_v), (stz, pz_v)):
        for j in range(VB):
            sl = pl.ds(j * 128, 128)
            pltpu.make_async_copy(st.at[vid_v.at[sl]], dst.at[sl], sem).wait()

    for sh, src in ((spx, px_v), (spy, py_v), (spz, pz_v)):
        pltpu.sync_copy(src, sh.at[pl.ds(s * VC, VC)])

    @pl.when((s % 2) == c)
    def _():
        for dst, src in ((px_o, px_v), (py_o, py_v), (pz_o, pz_v)):
            pltpu.sync_copy(src, dst.at[pl.ds(s * VC, VC)])

    plsc.subcore_barrier()
    scope1.__exit__(None, None, None)

    scope2 = jax.named_scope("ph2_faces")
    scope2.__enter__()
    def drain_adds(bsem):
        for _ in range(9):
            pltpu.make_async_copy(cx_v.at[pl.ds(0, 128)], sax.at[bA0], bsem).wait()

    def one_batch(b, bufs, bsem):
        b0_v, b1_v, b2_v = bufs
        for fv, bv in ((f0_v, b0_v), (f1_v, b1_v), (f2_v, b2_v)):
            for q in range(128 // L):
                bv[pl.ds(q * L, L)] = fv[pl.ds(b * 128 + q * L, L)]
        gathers = ((b0_v, (v0x, v0y, v0z)),
                   (b1_v, (v1x, v1y, v1z)),
                   (b2_v, (v2x, v2y, v2z)))
        for bv, dsts in gathers:
            for sh, dst in zip((spx, spy, spz), dsts):
                pltpu.async_copy(sh.at[bv], dst, sem)
        for bv, dsts in gathers:
            for sh, dst in zip((spx, spy, spz), dsts):
                pltpu.make_async_copy(sh.at[bv], dst, sem).wait()
        for g in range(128 // L):
            sl = pl.ds(g * L, L)
            so = pl.ds(b * 128 + g * L, L)
            ax, ay, az = v0x[sl], v0y[sl], v0z[sl]
            e1x, e1y, e1z = v1x[sl] - ax, v1y[sl] - ay, v1z[sl] - az
            e2x, e2y, e2z = v2x[sl] - ax, v2y[sl] - ay, v2z[sl] - az
            cx_v[so] = e1y * e2z - e1z * e2y
            cy_v[so] = e1z * e2x - e1x * e2z
            cz_v[so] = e1x * e2y - e1y * e2x
        slb = pl.ds(b * 128, 128)
        for bv in (b0_v, b1_v, b2_v):
            for sh, src in ((sax, cx_v), (say, cy_v), (saz, cz_v)):
                pltpu.async_copy(src.at[slb], sh.at[bv], bsem, add=True)

    def face_pair(t, carry):
        @pl.when(t > 0)
        def _():
            drain_adds(semA)
        one_batch(2 * t, (bA0, bA1, bA2), semA)

        @pl.when(t > 0)
        def _():
            drain_adds(semB)
        one_batch(2 * t + 1, (bB0, bB1, bB2), semB)
        return carry

    lax.fori_loop(0, FB // 2, face_pair, 0)
    drain_adds(semA)
    one_batch(FB - 1, (bA0, bA1, bA2), semA)
    drain_adds(semA)
    drain_adds(semB)
    plsc.subcore_barrier()
    scope2.__exit__(None, None, None)

    scope3 = jax.named_scope("ph3_out")
    scope3.__enter__()
    sl3 = pl.ds(s * VC, VC)

    @pl.when(c == 0)
    def _():
        for sh, buf, dst in ((sax, ax_v, a0x_o), (say, ay_v, a0y_o), (saz, az_v, a0z_o)):
            pltpu.sync_copy(sh.at[sl3], buf)
            pltpu.sync_copy(buf, dst.at[sl3])

    @pl.when(c == 1)
    def _():
        for sh, buf, dst in ((sax, ax_v, a1x_o), (say, ay_v, a1y_o), (saz, az_v, a1z_o)):
            pltpu.sync_copy(sh.at[sl3], buf)
            pltpu.sync_copy(buf, dst.at[sl3])

    scope3.__exit__(None, None, None)


def _norm_tc(a0x, a0y, a0z, a1x, a1y, a1z, ox, oy, oz):
    x = a0x[...] + a1x[...]
    y = a0y[...] + a1y[...]
    z = a0z[...] + a1z[...]
    d = x * x + y * y + z * z
    ok = d > 1e-20
    n = jnp.maximum(jnp.sqrt(d), 1e-12)
    ox[...] = jnp.where(ok, x / n, 0.0)
    oy[...] = jnp.where(ok, y / n, 0.0)
    oz[...] = jnp.where(ok, z / n, 1.0)


@jax.jit
def kernel(tet_v, surface_vid, surface_f):
    tpad = jnp.zeros((NT_PAD - N_TET_V,), jnp.float32)
    tx = jnp.concatenate([tet_v[:, 0], tpad])
    ty = jnp.concatenate([tet_v[:, 1], tpad])
    tz = jnp.concatenate([tet_v[:, 2], tpad])

    vid = surface_vid.astype(jnp.int32)
    vid = jnp.concatenate([vid, jnp.zeros((NV_PAD - NV,), jnp.int32)])

    f32i = surface_f.astype(jnp.int32)
    pad = jnp.full((NF_PAD - NF,), NV, jnp.int32)
    f0 = jnp.concatenate([f32i[:, 0], pad])
    f1 = jnp.concatenate([f32i[:, 1], pad])
    f2 = jnp.concatenate([f32i[:, 2], pad])

    zer = jnp.zeros((VC,), jnp.float32)

    plane = jax.ShapeDtypeStruct((NV_PAD,), jnp.float32)
    vmemf = lambda n: pltpu.VMEM((n,), jnp.float32)
    vmemi = lambda n: pltpu.VMEM((n,), jnp.int32)
    shmf = lambda n: pltpu.VMEM_SHARED((n,), jnp.float32)
    run = pl.kernel(
        _body,
        out_type=(plane,) * 9,
        mesh=plsc.VectorSubcoreMesh(core_axis_name="c", subcore_axis_name="s"),
        scratch_types=[
            vmemf(TC_), vmemi(VC),
            vmemf(VC), vmemf(VC), vmemf(VC),
            vmemi(FC), vmemi(FC), vmemi(FC),
            vmemi(128), vmemi(128), vmemi(128),
            vmemi(128), vmemi(128), vmemi(128),
            vmemf(128), vmemf(128), vmemf(128),
            vmemf(128), vmemf(128), vmemf(128),
            vmemf(128), vmemf(128), vmemf(128),
            vmemf(FC), vmemf(FC), vmemf(FC),
            vmemf(VC), vmemf(VC), vmemf(VC),
            shmf(NT_PAD), shmf(NT_PAD), shmf(NT_PAD),
            shmf(NV_PAD), shmf(NV_PAD), shmf(NV_PAD),
            shmf(NV_PAD), shmf(NV_PAD), shmf(NV_PAD),
            pltpu.SemaphoreType.DMA,
            pltpu.SemaphoreType.DMA,
            pltpu.SemaphoreType.DMA,
        ],
    )
    px, py, pz, a0x, a0y, a0z, a1x, a1y, a1z = run(
        tx, ty, tz, vid, f0, f1, f2, zer)

    blk = jax.ShapeDtypeStruct((NV_PAD // 128, 128), jnp.float32)
    nx, ny, nz = pl.pallas_call(
        _norm_tc,
        out_shape=(blk,) * 3,
    )(*(a.reshape(NV_PAD // 128, 128) for a in (a0x, a0y, a0z, a1x, a1y, a1z)))

    v_pos = jnp.stack([px[:NV], py[:NV], pz[:NV]], axis=1)
    v_nrm = jnp.stack([nx.reshape(-1)[:NV],
                       ny.reshape(-1)[:NV],
                       nz.reshape(-1)[:NV]], axis=1)
    return v_pos, v_nrm


# speedup vs baseline: 1.5752x; 1.0275x over previous
import jax
import jax.numpy as jnp
from jax import lax
from jax.experimental import pallas as pl
from jax.experimental.pallas import tpu as pltpu
from jax.experimental.pallas import tpu_sc as plsc

N_TET_V = 100000
NV = 50000
NF = 100000

NS = 16
L = 16

TC_ = 6256
NT_PAD = NS * TC_

VC = 3200
NV_PAD = NS * VC
VB = VC // 128

FC = 3200
NF_PAD = 32 * FC
FB = FC // 128


def _body(tx_hbm, ty_hbm, tz_hbm, vid_hbm, f0_hbm, f1_hbm, f2_hbm,
          px_o, py_o, pz_o, a0x_o, a0y_o, a0z_o, a1x_o, a1y_o, a1z_o,
          tb_v, vid_v, px_v, py_v, pz_v,
          f0_v, f1_v, f2_v, bA0, bA1, bA2, bB0, bB1, bB2,
          v0x, v0y, v0z, v1x, v1y, v1z, v2x, v2y, v2z,
          cx_v, cy_v, cz_v, ax_v, ay_v, az_v,
          stx, sty, stz, spx, spy, spz, sax, say, saz,
          sem, semA, semB):
    c = lax.axis_index("c")
    s = lax.axis_index("s")

    scope0 = jax.named_scope("ph0_stage")
    scope0.__enter__()
    zv = jnp.zeros((L,), jnp.float32)

    def zloop(q, carry):
        ax_v[pl.ds(q * L, L)] = zv
        return carry

    lax.fori_loop(0, VC // L, zloop, 0)
    for sh in (sax, say, saz):
        pltpu.sync_copy(ax_v, sh.at[pl.ds(s * VC, VC)])

    slt = pl.ds(s * TC_, TC_)
    for th, st in ((tx_hbm, stx), (ty_hbm, sty), (tz_hbm, stz)):
        pltpu.sync_copy(th.at[slt], tb_v)
        pltpu.sync_copy(tb_v, st.at[slt])

    pltpu.sync_copy(vid_hbm.at[pl.ds(s * VC, VC)], vid_v)

    wid = c * NS + s
    for fh, fv in ((f0_hbm, f0_v), (f1_hbm, f1_v), (f2_hbm, f2_v)):
        pltpu.sync_copy(fh.at[pl.ds(wid * FC, FC)], fv)

    plsc.subcore_barrier()
    scope0.__exit__(None, None, None)

    scope1 = jax.named_scope("ph1_vpos")
    scope1.__enter__()
    for st, dst in ((stx, px_v), (sty, py_v), (stz, pz_v)):
        for j in range(VB):
            sl = pl.ds(j * 128, 128)
            pltpu.async_copy(st.at[vid_v.at[sl]], dst.at[sl], sem)
    for st, dst in ((stx, px_v), (sty, py_v), (stz, pz_v)):
        for j in range(VB):
            sl = pl.ds(j * 128, 128)
            pltpu.make_async_copy(st.at[vid_v.at[sl]], dst.at[sl], sem).wait()

    for sh, src in ((spx, px_v), (spy, py_v), (spz, pz_v)):
        pltpu.sync_copy(src, sh.at[pl.ds(s * VC, VC)])

    @pl.when((s % 2) == c)
    def _():
        for dst, src in ((px_o, px_v), (py_o, py_v), (pz_o, pz_v)):
            pltpu.sync_copy(src, dst.at[pl.ds(s * VC, VC)])

    plsc.subcore_barrier()
    scope1.__exit__(None, None, None)

    scope2 = jax.named_scope("ph2_faces")
    scope2.__enter__()
    def drain_adds(bsem):
        for _ in range(9):
            pltpu.make_async_copy(cx_v.at[pl.ds(0, 128)], sax.at[bA0], bsem).wait()

    def one_batch(b, bufs, bsem):
        b0_v, b1_v, b2_v = bufs
        for fv, bv in ((f0_v, b0_v), (f1_v, b1_v), (f2_v, b2_v)):
            for q in range(128 // L):
                bv[pl.ds(q * L, L)] = fv[pl.ds(b * 128 + q * L, L)]
        gathers = ((b0_v, (v0x, v0y, v0z)),
                   (b1_v, (v1x, v1y, v1z)),
                   (b2_v, (v2x, v2y, v2z)))
        for bv, dsts in gathers:
            for sh, dst in zip((spx, spy, spz), dsts):
                pltpu.async_copy(sh.at[bv], dst, sem)
        for bv, dsts in gathers:
            for sh, dst in zip((spx, spy, spz), dsts):
                pltpu.make_async_copy(sh.at[bv], dst, sem).wait()
        for g in range(128 // L):
            sl = pl.ds(g * L, L)
            so = pl.ds(b * 128 + g * L, L)
            ax, ay, az = v0x[sl], v0y[sl], v0z[sl]
            e1x, e1y, e1z = v1x[sl] - ax, v1y[sl] - ay, v1z[sl] - az
            e2x, e2y, e2z = v2x[sl] - ax, v2y[sl] - ay, v2z[sl] - az
            cx_v[so] = e1y * e2z - e1z * e2y
            cy_v[so] = e1z * e2x - e1x * e2z
            cz_v[so] = e1x * e2y - e1y * e2x
        slb = pl.ds(b * 128, 128)
        for bv in (b0_v, b1_v, b2_v):
            for sh, src in ((sax, cx_v), (say, cy_v), (saz, cz_v)):
                pltpu.async_copy(src.at[slb], sh.at[bv], bsem, add=True)

    def face_pair(t, carry):
        @pl.when(t > 0)
        def _():
            drain_adds(semA)
        one_batch(2 * t, (bA0, bA1, bA2), semA)

        @pl.when(t > 0)
        def _():
            drain_adds(semB)
        one_batch(2 * t + 1, (bB0, bB1, bB2), semB)
        return carry

    lax.fori_loop(0, FB // 2, face_pair, 0)
    drain_adds(semA)
    one_batch(FB - 1, (bA0, bA1, bA2), semA)
    drain_adds(semA)
    drain_adds(semB)
    plsc.subcore_barrier()
    scope2.__exit__(None, None, None)

    scope3 = jax.named_scope("ph3_out")
    scope3.__enter__()
    sl3 = pl.ds(s * VC, VC)

    @pl.when(c == 0)
    def _():
        for sh, buf, dst in ((sax, ax_v, a0x_o), (say, ay_v, a0y_o), (saz, az_v, a0z_o)):
            pltpu.sync_copy(sh.at[sl3], buf)
            pltpu.sync_copy(buf, dst.at[sl3])

    @pl.when(c == 1)
    def _():
        for sh, buf, dst in ((sax, ax_v, a1x_o), (say, ay_v, a1y_o), (saz, az_v, a1z_o)):
            pltpu.sync_copy(sh.at[sl3], buf)
            pltpu.sync_copy(buf, dst.at[sl3])

    scope3.__exit__(None, None, None)


def _norm_tc(a0x, a0y, a0z, a1x, a1y, a1z, ox, oy, oz):
    x = a0x[...] + a1x[...]
    y = a0y[...] + a1y[...]
    z = a0z[...] + a1z[...]
    d = x * x + y * y + z * z
    ok = d > 1e-20
    n = jnp.maximum(jnp.sqrt(d), 1e-12)
    ox[...] = jnp.where(ok, x / n, 0.0)
    oy[...] = jnp.where(ok, y / n, 0.0)
    oz[...] = jnp.where(ok, z / n, 1.0)


@jax.jit
def kernel(tet_v, surface_vid, surface_f):
    tpad = jnp.zeros((NT_PAD - N_TET_V,), jnp.float32)
    tx = jnp.concatenate([tet_v[:, 0], tpad])
    ty = jnp.concatenate([tet_v[:, 1], tpad])
    tz = jnp.concatenate([tet_v[:, 2], tpad])

    vid = surface_vid.astype(jnp.int32)
    vid = jnp.concatenate([vid, jnp.zeros((NV_PAD - NV,), jnp.int32)])

    f32i = surface_f.astype(jnp.int32)
    pad = jnp.full((NF_PAD - NF,), NV, jnp.int32)
    f0 = jnp.concatenate([f32i[:, 0], pad])
    f1 = jnp.concatenate([f32i[:, 1], pad])
    f2 = jnp.concatenate([f32i[:, 2], pad])

    plane = jax.ShapeDtypeStruct((NV_PAD,), jnp.float32)
    vmemf = lambda n: pltpu.VMEM((n,), jnp.float32)
    vmemi = lambda n: pltpu.VMEM((n,), jnp.int32)
    shmf = lambda n: pltpu.VMEM_SHARED((n,), jnp.float32)
    run = pl.kernel(
        _body,
        out_type=(plane,) * 9,
        mesh=plsc.VectorSubcoreMesh(core_axis_name="c", subcore_axis_name="s"),
        scratch_types=[
            vmemf(TC_), vmemi(VC),
            vmemf(VC), vmemf(VC), vmemf(VC),
            vmemi(FC), vmemi(FC), vmemi(FC),
            vmemi(128), vmemi(128), vmemi(128),
            vmemi(128), vmemi(128), vmemi(128),
            vmemf(128), vmemf(128), vmemf(128),
            vmemf(128), vmemf(128), vmemf(128),
            vmemf(128), vmemf(128), vmemf(128),
            vmemf(FC), vmemf(FC), vmemf(FC),
            vmemf(VC), vmemf(VC), vmemf(VC),
            shmf(NT_PAD), shmf(NT_PAD), shmf(NT_PAD),
            shmf(NV_PAD), shmf(NV_PAD), shmf(NV_PAD),
            shmf(NV_PAD), shmf(NV_PAD), shmf(NV_PAD),
            pltpu.SemaphoreType.DMA,
            pltpu.SemaphoreType.DMA,
            pltpu.SemaphoreType.DMA,
        ],
    )
    px, py, pz, a0x, a0y, a0z, a1x, a1y, a1z = run(
        tx, ty, tz, vid, f0, f1, f2)

    blk = jax.ShapeDtypeStruct((NV_PAD // 128, 128), jnp.float32)
    nx, ny, nz = pl.pallas_call(
        _norm_tc,
        out_shape=(blk,) * 3,
    )(*(a.reshape(NV_PAD // 128, 128) for a in (a0x, a0y, a0z, a1x, a1y, a1z)))

    v_pos = jnp.stack([px[:NV], py[:NV], pz[:NV]], axis=1)
    v_nrm = jnp.stack([nx.reshape(-1)[:NV],
                       ny.reshape(-1)[:NV],
                       nz.reshape(-1)[:NV]], axis=1)
    return v_pos, v_nrm
